# baseline (speedup 1.0000x reference)
import functools

import jax
import jax.numpy as jnp
from jax import lax
from jax.experimental import pallas as pl
from jax.experimental.pallas import tpu as pltpu
from jax.experimental.pallas import tpu_sc as plsc

N_STATES = 1000000
OUTPUT_DIM = 16
BATCH = 16384

_info = plsc.get_sparse_core_info()
_NC, _NS = _info.num_cores, _info.num_subcores
_NW = _NC * _NS
_B_PER_W = BATCH // _NW
_CHUNK = 128
_NCHUNK = _B_PER_W // _CHUNK

_mesh = plsc.VectorSubcoreMesh(core_axis_name="c", subcore_axis_name="s")


@functools.partial(
    pl.kernel,
    mesh=_mesh,
    out_type=jax.ShapeDtypeStruct((BATCH, OUTPUT_DIM), jnp.float32),
    scratch_types=[
        pltpu.VMEM((_NCHUNK, _CHUNK), jnp.int32),
        pltpu.VMEM((_B_PER_W, OUTPUT_DIM), jnp.float32),
        pltpu.SemaphoreType.DMA,
    ],
    compiler_params=pltpu.CompilerParams(use_tc_tiling_on_sc=False),
)
def _gather_kernel(idx_hbm, table_hbm, out_hbm, idx_v, rows_v, sem):
    wid = lax.axis_index("s") * _NC + lax.axis_index("c")
    pltpu.sync_copy(idx_hbm.at[wid], idx_v)
    handles = [
        pltpu.async_copy(
            table_hbm.at[idx_v.at[j]],
            rows_v.at[pl.ds(j * _CHUNK, _CHUNK)],
            sem,
        )
        for j in range(_NCHUNK)
    ]
    for h in handles:
        h.wait()
    pltpu.sync_copy(rows_v, out_hbm.at[pl.ds(wid * _B_PER_W, _B_PER_W)])


def kernel(preprocessed_states, table):
    idx = preprocessed_states.reshape(_NW, _NCHUNK, _CHUNK)
    return _gather_kernel(idx, table)


# speedup vs baseline: 4.9326x; 4.9326x over previous
import functools

import jax
import jax.numpy as jnp
from jax import lax
from jax.experimental import pallas as pl
from jax.experimental.pallas import tpu as pltpu
from jax.experimental.pallas import tpu_sc as plsc

N_STATES = 1000000
OUTPUT_DIM = 16
BATCH = 16384

_info = plsc.get_sparse_core_info()
_NC, _NS = _info.num_cores, _info.num_subcores
_NW = _NC * _NS
_B_PER_W = BATCH // _NW
_VB = 16
_NB = _B_PER_W // _VB

_mesh = plsc.VectorSubcoreMesh(core_axis_name="c", subcore_axis_name="s")


@functools.partial(
    pl.kernel,
    mesh=_mesh,
    out_type=jax.ShapeDtypeStruct((OUTPUT_DIM, BATCH), jnp.float32),
    scratch_types=[
        pltpu.VMEM((_B_PER_W,), jnp.int32),
        pltpu.VMEM((_VB, OUTPUT_DIM, 128), jnp.float32),
        pltpu.VMEM((OUTPUT_DIM, _B_PER_W), jnp.float32),
        pltpu.SemaphoreType.DMA,
    ],
    compiler_params=pltpu.CompilerParams(needs_layout_passes=False),
)
def _gather_kernel(table_t_hbm, idx_hbm, out_t_hbm, idx_v, bufs_v, stage_v, sem):
    wid = lax.axis_index("s") * _NC + lax.axis_index("c")
    base = wid * _B_PER_W
    pltpu.sync_copy(idx_hbm.at[pl.ds(base, _B_PER_W)], idx_v)

    rows = lax.iota(jnp.int32, 16)
    half = rows >> 3
    frow = rows & 7

    def batch_body(g, carry):
        iv = idx_v[pl.ds(g * _VB, _VB)]
        handles = []
        for k in range(_VB):
            s_k = iv[k]
            g128 = pl.multiple_of((s_k >> 7) * 128, 128)
            handles.append(pltpu.async_copy(
                table_t_hbm.at[:, pl.ds(g128, 128)],
                bufs_v.at[k],
                sem,
            ))
        for h_ in handles:
            h_.wait()
        for k in range(_VB):
            s_k = iv[k]
            col = s_k & 127
            vals = plsc.load_gather(bufs_v.at[k], [rows, jnp.full((16,), col, jnp.int32)])
            h = g * _VB + k
            plsc.store_scatter(stage_v, [rows, jnp.full((16,), h, jnp.int32)], vals)
        return carry

    lax.fori_loop(0, _NB, batch_body, 0)
    pltpu.sync_copy(stage_v, out_t_hbm.at[:, pl.ds(base, _B_PER_W)])


def kernel(preprocessed_states, table):
    idx = preprocessed_states.reshape(BATCH)
    out_t = _gather_kernel(table.T, idx)
    return out_t.T


# speedup vs baseline: 5.9529x; 1.2068x over previous
import functools

import jax
import jax.numpy as jnp
from jax import lax
from jax.experimental import pallas as pl
from jax.experimental.pallas import tpu as pltpu
from jax.experimental.pallas import tpu_sc as plsc

N_STATES = 1000000
OUTPUT_DIM = 16
BATCH = 16384

_info = plsc.get_sparse_core_info()
_NC, _NS = _info.num_cores, _info.num_subcores
_NW = _NC * _NS
_B_PER_W = BATCH // _NW
_VB = 16
_NB = _B_PER_W // _VB

_mesh = plsc.VectorSubcoreMesh(core_axis_name="c", subcore_axis_name="s")


@functools.partial(
    pl.kernel,
    mesh=_mesh,
    out_type=jax.ShapeDtypeStruct((OUTPUT_DIM, BATCH), jnp.float32),
    scratch_types=[
        pltpu.VMEM((_B_PER_W,), jnp.int32),
        pltpu.VMEM((_VB, OUTPUT_DIM, 128), jnp.float32),
        pltpu.VMEM((_VB, OUTPUT_DIM, 128), jnp.float32),
        pltpu.VMEM((OUTPUT_DIM, _B_PER_W), jnp.float32),
        pltpu.SemaphoreType.DMA,
        pltpu.SemaphoreType.DMA,
    ],
    compiler_params=pltpu.CompilerParams(needs_layout_passes=False),
)
def _gather_kernel(table_t_hbm, idx_hbm, out_t_hbm,
                   idx_v, bufs0, bufs1, stage_v, sem0, sem1):
    wid = lax.axis_index("s") * _NC + lax.axis_index("c")
    base = wid * _B_PER_W
    pltpu.sync_copy(idx_hbm.at[pl.ds(base, _B_PER_W)], idx_v)

    rows = lax.iota(jnp.int32, 16)

    def fire(g, bufs, sem):
        iv = idx_v[pl.ds(g * _VB, _VB)]
        for k in range(_VB):
            g128 = pl.multiple_of((iv[k] >> 7) * 128, 128)
            pltpu.async_copy(
                table_t_hbm.at[:, pl.ds(g128, 128)], bufs.at[k], sem)

    def drain(bufs, sem):
        for k in range(_VB):
            pltpu.make_async_copy(
                table_t_hbm.at[:, pl.ds(0, 128)], bufs.at[k], sem).wait()

    def extract(g, bufs):
        iv = idx_v[pl.ds(g * _VB, _VB)]
        for k in range(_VB):
            s_k = iv[k]
            vals = plsc.load_gather(
                bufs.at[k], [rows, jnp.full((16,), s_k & 127, jnp.int32)])
            plsc.store_scatter(
                stage_v,
                [rows, jnp.full((16,), g * _VB + k, jnp.int32)],
                vals)

    fire(0, bufs0, sem0)

    def body(j, carry):
        g0 = j * 2
        fire(g0 + 1, bufs1, sem1)
        drain(bufs0, sem0)
        extract(g0, bufs0)

        @pl.when(j < _NB // 2 - 1)
        def _():
            fire(g0 + 2, bufs0, sem0)

        drain(bufs1, sem1)
        extract(g0 + 1, bufs1)
        return carry

    lax.fori_loop(0, _NB // 2, body, 0)
    pltpu.sync_copy(stage_v, out_t_hbm.at[:, pl.ds(base, _B_PER_W)])


def kernel(preprocessed_states, table):
    idx = preprocessed_states.reshape(BATCH)
    out_t = _gather_kernel(table.T, idx)
    return out_t.T
